# Initial kernel scaffold; baseline (speedup 1.0000x reference)
import jax
import jax.numpy as jnp
from jax.experimental import pallas as pl


def kernel(anchors, gt_bboxes, gt_orig_classes):
    raise NotImplementedError("write your pallas kernel here")


# speedup vs baseline: 3.8771x; 3.8771x over previous
import functools

import jax
import jax.numpy as jnp
from jax import lax
from jax.experimental import pallas as pl
from jax.experimental.pallas import tpu as pltpu
from jax.experimental.pallas import tpu_sc as plsc

FM_W, FM_H = 50.0, 50.0
TM_W, TM_H = 800.0, 800.0
POS_TH = 0.7
NEG_TH = 0.3
MIN_POS_TH = 0.5 * NEG_TH
RATIO = 0.5

PCAP = 10240
LN = 16
IB = 128
JB = 2048
CHUNK = 8000


def _iou_code_body(x1, y1, x2, y2, g1, g2, g3, g4, iou_ref, code_ref):
    ax1, ay1, ax2, ay2 = x1[0], y1[0], x2[0], y2[0]
    sx = FM_W / TM_W
    sy = FM_H / TM_H
    bx1 = g1[0] * sx
    by1 = g2[0] * sy
    bx2 = g3[0] * sx
    by2 = g4[0] * sy
    area_a = (ax2 - ax1) * (ay2 - ay1)
    area_b = (bx2 - bx1) * (by2 - by1)
    w = jnp.maximum(jnp.minimum(ax2, bx2) - jnp.maximum(ax1, bx1), 0.0)
    h = jnp.maximum(jnp.minimum(ay2, by2) - jnp.maximum(ay1, by1), 0.0)
    inter = w * h
    iou = inter / (area_a + area_b - inter)
    valid = (ax1 >= 0) & (ay1 >= 0) & (ax2 <= FM_W) & (ay2 <= FM_H)
    rowmax = jnp.max(iou, axis=0, keepdims=True)
    pos = (((iou == rowmax) & (iou > MIN_POS_TH)) | (iou >= POS_TH)) & valid
    neg = (iou < NEG_TH) & (~pos) & valid
    code = jnp.where(pos, 1, jnp.where(neg, 2, 0)).astype(jnp.int32)
    iou_ref[0] = iou
    code_ref[0] = code


def _run_iou_code(B, A, G, comps, gcols):
    return pl.pallas_call(
        _iou_code_body,
        grid=(B,),
        in_specs=[pl.BlockSpec((1, 1, A), lambda b: (b, 0, 0))] * 4
        + [pl.BlockSpec((1, G, 1), lambda b: (b, 0, 0))] * 4,
        out_specs=[pl.BlockSpec((1, G, A), lambda b: (b, 0, 0))] * 2,
        out_shape=[
            jax.ShapeDtypeStruct((B, G, A), jnp.float32),
            jax.ShapeDtypeStruct((B, G, A), jnp.int32),
        ],
    )(*comps, *gcols)


def _make_compact(B, N, negcap):
    nch = N // CHUNK
    info = plsc.get_sparse_core_info()
    nc = info.num_cores
    mesh = plsc.VectorSubcoreMesh(core_axis_name="c", subcore_axis_name="s")

    @functools.partial(
        pl.kernel,
        mesh=mesh,
        out_type=[
            jax.ShapeDtypeStruct((B * PCAP,), jnp.float32),
            jax.ShapeDtypeStruct((B * PCAP,), jnp.int32),
            jax.ShapeDtypeStruct((B * negcap,), jnp.int32),
            jax.ShapeDtypeStruct((B * LN,), jnp.int32),
        ],
        scratch_types=[
            pltpu.VMEM((CHUNK,), jnp.float32),
            pltpu.VMEM((CHUNK,), jnp.int32),
            pltpu.VMEM((PCAP + LN,), jnp.float32),
            pltpu.VMEM((PCAP + LN,), jnp.int32),
            pltpu.VMEM((negcap + LN,), jnp.int32),
            pltpu.VMEM((LN,), jnp.int32),
        ],
        compiler_params=pltpu.CompilerParams(needs_layout_passes=False),
    )
    def k2(iou_hbm, code_hbm, piou_hbm, pidx_hbm, neg_hbm, cnt_hbm,
           iou_c, code_c, piou_v, pidx_v, neg_v, cnt_v):
        wid = lax.axis_index("s") * nc + lax.axis_index("c")

        @pl.when(wid < B)
        def _():
            b = wid
            iota = lax.iota(jnp.int32, LN)

            def initbody(k, _):
                piou_v[pl.ds(k * LN, LN)] = jnp.full((LN,), -1.0, jnp.float32)
                pidx_v[pl.ds(k * LN, LN)] = jnp.zeros((LN,), jnp.int32)
                return 0

            lax.fori_loop(0, (PCAP + LN) // LN, initbody, 0)

            def chunk(c, carry):
                src_off = pl.multiple_of(b * N + c * CHUNK, 8)
                pltpu.sync_copy(iou_hbm.at[pl.ds(src_off, CHUNK)], iou_c)
                pltpu.sync_copy(code_hbm.at[pl.ds(src_off, CHUNK)], code_c)

                def step(k, cc):
                    cp, cn = cc
                    off = k * LN
                    iv = iou_c[pl.ds(off, LN)]
                    cv = code_c[pl.ds(off, LN)]
                    idxv = c * CHUNK + off + iota
                    mpos = cv == 1
                    mneg = cv == 2
                    ppref = plsc.cumsum(mpos.astype(jnp.int32))
                    pdest = cp + ppref - 1
                    plsc.store_scatter(piou_v, [pdest], iv, mask=mpos)
                    plsc.store_scatter(pidx_v, [pdest], idxv, mask=mpos)
                    cp = jnp.minimum(cp + jnp.max(ppref), PCAP)

                    npref = plsc.cumsum(mneg.astype(jnp.int32))

                    @pl.when(cn < negcap)
                    def _():
                        plsc.store_scatter(neg_v, [cn + npref - 1], idxv, mask=mneg)

                    cn = jnp.minimum(cn + jnp.max(npref), negcap)
                    return (cp, cn)

                return lax.fori_loop(0, CHUNK // LN, step, carry)

            cp, cn = lax.fori_loop(0, nch, chunk, (jnp.int32(0), jnp.int32(0)))
            cnt_v[pl.ds(0, LN)] = jnp.where(
                iota == 0, cp, jnp.where(iota == 1, cn, 0))
            pltpu.sync_copy(piou_v.at[pl.ds(0, PCAP)],
                            piou_hbm.at[pl.ds(pl.multiple_of(b * PCAP, 8), PCAP)])
            pltpu.sync_copy(pidx_v.at[pl.ds(0, PCAP)],
                            pidx_hbm.at[pl.ds(pl.multiple_of(b * PCAP, 8), PCAP)])
            pltpu.sync_copy(neg_v.at[pl.ds(0, negcap)],
                            neg_hbm.at[pl.ds(pl.multiple_of(b * negcap, 8), negcap)])
            pltpu.sync_copy(cnt_v, cnt_hbm.at[pl.ds(pl.multiple_of(b * LN, 8), LN)])

    return k2


def _rank_body(iouc, iour, out_ref):
    i = pl.program_id(1)
    j = pl.program_id(2)
    ivals = iouc[0]
    jvals = iour[0, 0]
    ig = lax.broadcasted_iota(jnp.int32, (IB, JB), 0) + i * IB
    jg = lax.broadcasted_iota(jnp.int32, (IB, JB), 1) + j * JB
    before = (jvals > ivals) | ((jvals == ivals) & (jg < ig))
    acc = jnp.sum(before.astype(jnp.int32), axis=1, keepdims=True)

    @pl.when(j == 0)
    def _():
        out_ref[0] = acc

    @pl.when(j != 0)
    def _():
        out_ref[0] += acc


def _run_rank(B, pos_iou):
    pos_iou_col = pos_iou[..., None]
    ranks = pl.pallas_call(
        _rank_body,
        grid=(B, PCAP // IB, PCAP // JB),
        in_specs=[
            pl.BlockSpec((1, IB, 1), lambda b, i, j: (b, i, 0)),
            pl.BlockSpec((1, 1, JB), lambda b, i, j: (b, 0, j)),
        ],
        out_specs=pl.BlockSpec((1, IB, 1), lambda b, i, j: (b, i, 0)),
        out_shape=jax.ShapeDtypeStruct((B, PCAP, 1), jnp.int32),
    )(pos_iou_col, pos_iou[:, None, :])
    return ranks.reshape(B, PCAP)


def _make_scatter_gather(B, A, G, negcap, num_pos):
    info = plsc.get_sparse_core_info()
    nc = info.num_cores
    mesh = plsc.VectorSubcoreMesh(core_axis_name="c", subcore_axis_name="s")
    shift = G.bit_length() - 1 if G & (G - 1) == 0 else None

    @functools.partial(
        pl.kernel,
        mesh=mesh,
        out_type=[jax.ShapeDtypeStruct((B * A,), jnp.float32)] * 4,
        scratch_types=[
            pltpu.VMEM((A + LN,), jnp.int32),
            pltpu.VMEM((PCAP,), jnp.int32),
            pltpu.VMEM((PCAP,), jnp.int32),
            pltpu.VMEM((negcap,), jnp.int32),
            pltpu.VMEM((A,), jnp.float32),
            pltpu.VMEM((A,), jnp.float32),
            pltpu.VMEM((LN,), jnp.int32),
        ],
        compiler_params=pltpu.CompilerParams(needs_layout_passes=False),
    )
    def k4(pidx_hbm, ranks_hbm, cnt_hbm, neg_hbm, x1_hbm, y1_hbm, x2_hbm,
           y2_hbm, o1_hbm, o2_hbm, o3_hbm, o4_hbm,
           order_v, ranks_v, pidx_v, neg_v, comp_v, gout_v, cnt_v):
        wid = lax.axis_index("s") * nc + lax.axis_index("c")

        @pl.when(wid < B)
        def _():
            b = wid
            iota = lax.iota(jnp.int32, LN)
            pltpu.sync_copy(cnt_hbm.at[pl.ds(pl.multiple_of(b * LN, 8), LN)], cnt_v)
            cvec = cnt_v[pl.ds(0, LN)]
            P = jnp.max(jnp.where(iota == 0, cvec, 0))
            S_p = jnp.minimum(P, num_pos)

            def zbody(k, _):
                order_v[pl.ds(k * LN, LN)] = jnp.zeros((LN,), jnp.int32)
                return 0

            lax.fori_loop(0, (A + LN) // LN, zbody, 0)
            pltpu.sync_copy(pidx_hbm.at[pl.ds(pl.multiple_of(b * PCAP, 8), PCAP)], pidx_v)
            pltpu.sync_copy(ranks_hbm.at[pl.ds(pl.multiple_of(b * PCAP, 8), PCAP)], ranks_v)
            pltpu.sync_copy(neg_hbm.at[pl.ds(pl.multiple_of(b * negcap, 8), negcap)], neg_v)

            def scat(k, _):
                off = k * LN
                rv = ranks_v[pl.ds(off, LN)]
                vv = pidx_v[pl.ds(off, LN)]
                m = ((off + iota) < P) & (rv < num_pos)
                plsc.store_scatter(order_v, [rv], vv, mask=m)
                return 0

            lax.fori_loop(0, PCAP // LN, scat, 0)

            ncopy = (A - S_p + (LN - 1)) // LN

            def negc(j, _):
                order_v[pl.ds(S_p + j * LN, LN)] = neg_v[pl.ds(j * LN, LN)]
                return 0

            lax.fori_loop(0, ncopy, negc, 0)

            for src, dst in ((x1_hbm, o1_hbm), (y1_hbm, o2_hbm),
                             (x2_hbm, o3_hbm), (y2_hbm, o4_hbm)):
                pltpu.sync_copy(src.at[pl.ds(pl.multiple_of(b * A, 8), A)], comp_v)

                def gbody(k, _):
                    ov = order_v[pl.ds(k * LN, LN)]
                    if shift is not None:
                        av = lax.shift_right_logical(ov, shift)
                    else:
                        av = ov // G
                    gout_v[pl.ds(k * LN, LN)] = plsc.load_gather(comp_v, [av])
                    return 0

                lax.fori_loop(0, A // LN, gbody, 0)
                pltpu.sync_copy(gout_v, dst.at[pl.ds(pl.multiple_of(b * A, 8), A)])

    return k4


def _make_offsets_body(num_pos, A):
    def body(x1, y1, x2, y2, b1, b2, b3, b4, cnt,
             tx_ref, ty_ref, tw_ref, th_ref, obj_ref):
        ax1, ay1, ax2, ay2 = x1[0], y1[0], x2[0], y2[0]
        acx = (ax1 + ax2) / 2.0
        acy = (ay1 + ay2) / 2.0
        aw = ax2 - ax1
        ah = ay2 - ay1
        bx1 = b1[0] * (FM_W / TM_W)
        by1 = b2[0] * (FM_H / TM_H)
        bx2 = b3[0] * (FM_W / TM_W)
        by2 = b4[0] * (FM_H / TM_H)
        bcx = (bx1 + bx2) / 2.0
        bcy = (by1 + by2) / 2.0
        bw = bx2 - bx1
        bh = by2 - by1
        tx_ref[0] = (bcx - acx) / aw
        ty_ref[0] = (bcy - acy) / ah
        tw_ref[0] = jnp.log(bw / aw)
        th_ref[0] = jnp.log(bh / ah)
        S_p = jnp.minimum(cnt[0], num_pos)
        pos_iota = lax.broadcasted_iota(jnp.int32, (1, A), 1)
        obj_ref[0] = (pos_iota < S_p).astype(jnp.float32)

    return body


def _run_offsets(B, A, num_pos, gcomps, b0cols, cnt_col):
    return pl.pallas_call(
        _make_offsets_body(num_pos, A),
        grid=(B,),
        in_specs=[pl.BlockSpec((1, 1, A), lambda b: (b, 0, 0))] * 4
        + [pl.BlockSpec((1, 1, 1), lambda b: (b, 0, 0))] * 5,
        out_specs=[pl.BlockSpec((1, 1, A), lambda b: (b, 0, 0))] * 5,
        out_shape=[jax.ShapeDtypeStruct((B, 1, A), jnp.float32)] * 5,
    )(*[g[:, None, :] for g in gcomps], *[c[:, :, None] for c in b0cols],
      cnt_col[:, :, None])


def kernel(anchors, gt_bboxes, gt_orig_classes):
    B, A, _ = anchors.shape
    G = gt_bboxes.shape[1]
    N = A * G
    num_neg = int(RATIO * A)
    num_pos = A - num_neg
    negcap = A + LN

    comps = [anchors[..., i] for i in range(4)]
    gcols = [gt_bboxes[..., i][..., None] for i in range(4)]

    comps3 = [c[:, None, :] for c in comps]
    iou_t, code_t = _run_iou_code(B, A, G, comps3, gcols)
    iou_flat = iou_t.transpose(0, 2, 1).reshape(B, N)
    code_flat = code_t.transpose(0, 2, 1).reshape(B, N)

    k2 = _make_compact(B, N, negcap)
    pos_iou, pos_idx, neg_idx, counts = k2(
        iou_flat.reshape(-1), code_flat.reshape(-1))
    pos_iou = pos_iou.reshape(B, PCAP)
    pos_idx = pos_idx.reshape(B, PCAP)
    neg_idx = neg_idx.reshape(B, negcap)
    counts = counts.reshape(B, LN)

    ranks = _run_rank(B, pos_iou)

    k4 = _make_scatter_gather(B, A, G, negcap, num_pos)
    g1, g2, g3, g4 = (g.reshape(B, A) for g in k4(
        pos_idx.reshape(-1), ranks.reshape(-1), counts.reshape(-1),
        neg_idx.reshape(-1), *[c.reshape(-1) for c in comps]))

    b0cols = [gt_bboxes[:, 0, i][:, None] for i in range(4)]
    cnt_col = counts[:, 0:1]
    tx, ty, tw, th, obj = (o[:, 0, :] for o in _run_offsets(
        B, A, num_pos, (g1, g2, g3, g4), b0cols, cnt_col))

    all_anchors = jnp.stack([g1, g2, g3, g4], axis=-1)
    gt_conf = obj[..., None]
    gt_cls = jnp.broadcast_to(
        gt_orig_classes[:, 0][:, None, None], (B, A, 1)
    ).astype(gt_orig_classes.dtype)
    gt_off = jnp.stack([tx, ty, tw, th], axis=-1)
    return all_anchors, gt_conf, gt_cls, gt_off


# speedup vs baseline: 4.4866x; 1.1572x over previous
import functools

import jax
import jax.numpy as jnp
from jax import lax
from jax.experimental import pallas as pl
from jax.experimental.pallas import tpu as pltpu
from jax.experimental.pallas import tpu_sc as plsc

FM_W, FM_H = 50.0, 50.0
TM_W, TM_H = 800.0, 800.0
POS_TH = 0.7
NEG_TH = 0.3
MIN_POS_TH = 0.5 * NEG_TH
RATIO = 0.5

PCAP = 10240
LN = 16
IB = 128
JB = 2048
CHUNK = 8000


def _iou_code_body(x1, y1, x2, y2, g1, g2, g3, g4, iou_ref, code_ref):
    ax1, ay1, ax2, ay2 = x1[0], y1[0], x2[0], y2[0]
    sx = FM_W / TM_W
    sy = FM_H / TM_H
    bx1 = g1[0] * sx
    by1 = g2[0] * sy
    bx2 = g3[0] * sx
    by2 = g4[0] * sy
    area_a = (ax2 - ax1) * (ay2 - ay1)
    area_b = (bx2 - bx1) * (by2 - by1)
    w = jnp.maximum(jnp.minimum(ax2, bx2) - jnp.maximum(ax1, bx1), 0.0)
    h = jnp.maximum(jnp.minimum(ay2, by2) - jnp.maximum(ay1, by1), 0.0)
    inter = w * h
    iou = inter / (area_a + area_b - inter)
    valid = (ax1 >= 0) & (ay1 >= 0) & (ax2 <= FM_W) & (ay2 <= FM_H)
    rowmax = jnp.max(iou, axis=0, keepdims=True)
    pos = (((iou == rowmax) & (iou > MIN_POS_TH)) | (iou >= POS_TH)) & valid
    neg = (iou < NEG_TH) & (~pos) & valid
    code = jnp.where(pos, 1, jnp.where(neg, 2, 0)).astype(jnp.int32)
    iou_ref[0] = iou
    code_ref[0] = code


def _run_iou_code(B, A, G, comps, gcols):
    return pl.pallas_call(
        _iou_code_body,
        grid=(B,),
        in_specs=[pl.BlockSpec((1, 1, A), lambda b: (b, 0, 0))] * 4
        + [pl.BlockSpec((1, G, 1), lambda b: (b, 0, 0))] * 4,
        out_specs=[pl.BlockSpec((1, G, A), lambda b: (b, 0, 0))] * 2,
        out_shape=[
            jax.ShapeDtypeStruct((B, G, A), jnp.float32),
            jax.ShapeDtypeStruct((B, G, A), jnp.int32),
        ],
    )(*comps, *gcols)


def _make_compact(B, N, negcap):
    nch = N // CHUNK
    info = plsc.get_sparse_core_info()
    nc = info.num_cores
    mesh = plsc.VectorSubcoreMesh(core_axis_name="c", subcore_axis_name="s")

    @functools.partial(
        pl.kernel,
        mesh=mesh,
        out_type=[
            jax.ShapeDtypeStruct((B * PCAP,), jnp.float32),
            jax.ShapeDtypeStruct((B * PCAP,), jnp.int32),
            jax.ShapeDtypeStruct((B * negcap,), jnp.int32),
            jax.ShapeDtypeStruct((B * LN,), jnp.int32),
        ],
        scratch_types=[
            pltpu.VMEM((CHUNK,), jnp.float32),
            pltpu.VMEM((CHUNK,), jnp.int32),
            pltpu.VMEM((PCAP + LN,), jnp.float32),
            pltpu.VMEM((PCAP + LN,), jnp.int32),
            pltpu.VMEM((negcap + LN,), jnp.int32),
            pltpu.VMEM((LN,), jnp.int32),
        ],
        compiler_params=pltpu.CompilerParams(needs_layout_passes=False),
    )
    def k2(iou_hbm, code_hbm, piou_hbm, pidx_hbm, neg_hbm, cnt_hbm,
           iou_c, code_c, piou_v, pidx_v, neg_v, cnt_v):
        wid = lax.axis_index("s") * nc + lax.axis_index("c")

        @pl.when(wid < B)
        def _():
            b = wid
            iota = lax.iota(jnp.int32, LN)

            def initbody(k, _):
                piou_v[pl.ds(k * LN, LN)] = jnp.full((LN,), -1.0, jnp.float32)
                pidx_v[pl.ds(k * LN, LN)] = jnp.zeros((LN,), jnp.int32)
                return 0

            lax.fori_loop(0, (PCAP + LN) // LN, initbody, 0)

            def chunk(c, carry):
                src_off = pl.multiple_of(b * N + c * CHUNK, 8)
                pltpu.sync_copy(iou_hbm.at[pl.ds(src_off, CHUNK)], iou_c)
                pltpu.sync_copy(code_hbm.at[pl.ds(src_off, CHUNK)], code_c)

                def pos_part(cp, off, cv, idxv):
                    iv = iou_c[pl.ds(off, LN)]
                    mpos = cv == 1
                    ppref = plsc.cumsum(mpos.astype(jnp.int32))
                    pdest = cp + ppref - 1
                    plsc.store_scatter(piou_v, [pdest], iv, mask=mpos)
                    plsc.store_scatter(pidx_v, [pdest], idxv, mask=mpos)
                    return jnp.minimum(cp + jnp.max(ppref), PCAP)

                def step_full(k, cc):
                    cp, cn = cc
                    off = k * LN
                    cv = code_c[pl.ds(off, LN)]
                    idxv = c * CHUNK + off + iota
                    cp = pos_part(cp, off, cv, idxv)
                    mneg = cv == 2
                    npref = plsc.cumsum(mneg.astype(jnp.int32))

                    @pl.when(cn < negcap)
                    def _():
                        plsc.store_scatter(neg_v, [cn + npref - 1], idxv, mask=mneg)

                    cn = jnp.minimum(cn + jnp.max(npref), negcap)
                    return (cp, cn)

                def step_pos(k, cc):
                    cp, cn = cc
                    off = k * LN
                    cv = code_c[pl.ds(off, LN)]
                    idxv = c * CHUNK + off + iota
                    return (pos_part(cp, off, cv, idxv), cn)

                return lax.cond(
                    carry[1] < negcap,
                    lambda cc: lax.fori_loop(0, CHUNK // LN, step_full, cc),
                    lambda cc: lax.fori_loop(0, CHUNK // LN, step_pos, cc),
                    carry)

            cp, cn = lax.fori_loop(0, nch, chunk, (jnp.int32(0), jnp.int32(0)))
            cnt_v[pl.ds(0, LN)] = jnp.where(
                iota == 0, cp, jnp.where(iota == 1, cn, 0))
            pltpu.sync_copy(piou_v.at[pl.ds(0, PCAP)],
                            piou_hbm.at[pl.ds(pl.multiple_of(b * PCAP, 8), PCAP)])
            pltpu.sync_copy(pidx_v.at[pl.ds(0, PCAP)],
                            pidx_hbm.at[pl.ds(pl.multiple_of(b * PCAP, 8), PCAP)])
            pltpu.sync_copy(neg_v.at[pl.ds(0, negcap)],
                            neg_hbm.at[pl.ds(pl.multiple_of(b * negcap, 8), negcap)])
            pltpu.sync_copy(cnt_v, cnt_hbm.at[pl.ds(pl.multiple_of(b * LN, 8), LN)])

    return k2


def _rank_body(iouc, iour, cnt, out_ref):
    i = pl.program_id(1)
    j = pl.program_id(2)
    P = jnp.max(cnt[0])
    live = (i * IB < P) & (j * JB < P)

    @pl.when(live)
    def _():
        ivals = iouc[0]
        jvals = iour[0, 0]
        ig = lax.broadcasted_iota(jnp.int32, (IB, JB), 0) + i * IB
        jg = lax.broadcasted_iota(jnp.int32, (IB, JB), 1) + j * JB
        before = (jvals > ivals) | ((jvals == ivals) & (jg < ig))
        acc = jnp.sum(before.astype(jnp.int32), axis=1, keepdims=True)

        @pl.when(j == 0)
        def _():
            out_ref[0] = acc

        @pl.when(j != 0)
        def _():
            out_ref[0] += acc

    @pl.when(jnp.logical_not(live) & (j == 0))
    def _():
        out_ref[0] = jnp.zeros((IB, 1), jnp.int32)


def _run_rank(B, pos_iou, counts):
    pos_iou_col = pos_iou[..., None]
    ranks = pl.pallas_call(
        _rank_body,
        grid=(B, PCAP // IB, PCAP // JB),
        in_specs=[
            pl.BlockSpec((1, IB, 1), lambda b, i, j: (b, i, 0)),
            pl.BlockSpec((1, 1, JB), lambda b, i, j: (b, 0, j)),
            pl.BlockSpec((1, 1, 1), lambda b, i, j: (b, 0, 0)),
        ],
        out_specs=pl.BlockSpec((1, IB, 1), lambda b, i, j: (b, i, 0)),
        out_shape=jax.ShapeDtypeStruct((B, PCAP, 1), jnp.int32),
    )(pos_iou_col, pos_iou[:, None, :], counts[:, 0:1, None])
    return ranks.reshape(B, PCAP)


def _make_scatter_gather(B, A, G, negcap, num_pos):
    info = plsc.get_sparse_core_info()
    nc = info.num_cores
    mesh = plsc.VectorSubcoreMesh(core_axis_name="c", subcore_axis_name="s")
    shift = G.bit_length() - 1 if G & (G - 1) == 0 else None

    @functools.partial(
        pl.kernel,
        mesh=mesh,
        out_type=[jax.ShapeDtypeStruct((B * A,), jnp.float32)] * 4,
        scratch_types=[
            pltpu.VMEM((A + LN,), jnp.int32),
            pltpu.VMEM((PCAP,), jnp.int32),
            pltpu.VMEM((PCAP,), jnp.int32),
            pltpu.VMEM((negcap,), jnp.int32),
            pltpu.VMEM((A,), jnp.float32),
            pltpu.VMEM((A,), jnp.float32),
            pltpu.VMEM((LN,), jnp.int32),
        ],
        compiler_params=pltpu.CompilerParams(needs_layout_passes=False),
    )
    def k4(pidx_hbm, ranks_hbm, cnt_hbm, neg_hbm, x1_hbm, y1_hbm, x2_hbm,
           y2_hbm, o1_hbm, o2_hbm, o3_hbm, o4_hbm,
           order_v, ranks_v, pidx_v, neg_v, comp_v, gout_v, cnt_v):
        wid = lax.axis_index("s") * nc + lax.axis_index("c")

        @pl.when(wid < B)
        def _():
            b = wid
            iota = lax.iota(jnp.int32, LN)
            pltpu.sync_copy(cnt_hbm.at[pl.ds(pl.multiple_of(b * LN, 8), LN)], cnt_v)
            cvec = cnt_v[pl.ds(0, LN)]
            P = jnp.max(jnp.where(iota == 0, cvec, 0))
            S_p = jnp.minimum(P, num_pos)

            def zbody(k, _):
                order_v[pl.ds(k * LN, LN)] = jnp.zeros((LN,), jnp.int32)
                return 0

            lax.fori_loop(0, (A + LN) // LN, zbody, 0)
            pltpu.sync_copy(pidx_hbm.at[pl.ds(pl.multiple_of(b * PCAP, 8), PCAP)], pidx_v)
            pltpu.sync_copy(ranks_hbm.at[pl.ds(pl.multiple_of(b * PCAP, 8), PCAP)], ranks_v)
            pltpu.sync_copy(neg_hbm.at[pl.ds(pl.multiple_of(b * negcap, 8), negcap)], neg_v)

            def scat(k, _):
                off = k * LN
                rv = ranks_v[pl.ds(off, LN)]
                vv = pidx_v[pl.ds(off, LN)]
                m = ((off + iota) < P) & (rv < num_pos)
                plsc.store_scatter(order_v, [rv], vv, mask=m)
                return 0

            lax.fori_loop(0, PCAP // LN, scat, 0)

            ncopy = (A - S_p + (LN - 1)) // LN

            def negc(j, _):
                order_v[pl.ds(S_p + j * LN, LN)] = neg_v[pl.ds(j * LN, LN)]
                return 0

            lax.fori_loop(0, ncopy, negc, 0)

            for src, dst in ((x1_hbm, o1_hbm), (y1_hbm, o2_hbm),
                             (x2_hbm, o3_hbm), (y2_hbm, o4_hbm)):
                pltpu.sync_copy(src.at[pl.ds(pl.multiple_of(b * A, 8), A)], comp_v)

                def gbody(k, _):
                    ov = order_v[pl.ds(k * LN, LN)]
                    if shift is not None:
                        av = lax.shift_right_logical(ov, shift)
                    else:
                        av = ov // G
                    gout_v[pl.ds(k * LN, LN)] = plsc.load_gather(comp_v, [av])
                    return 0

                lax.fori_loop(0, A // LN, gbody, 0)
                pltpu.sync_copy(gout_v, dst.at[pl.ds(pl.multiple_of(b * A, 8), A)])

    return k4


def _make_offsets_body(num_pos, A):
    def body(x1, y1, x2, y2, b1, b2, b3, b4, cnt,
             tx_ref, ty_ref, tw_ref, th_ref, obj_ref):
        ax1, ay1, ax2, ay2 = x1[0], y1[0], x2[0], y2[0]
        acx = (ax1 + ax2) / 2.0
        acy = (ay1 + ay2) / 2.0
        aw = ax2 - ax1
        ah = ay2 - ay1
        bx1 = b1[0] * (FM_W / TM_W)
        by1 = b2[0] * (FM_H / TM_H)
        bx2 = b3[0] * (FM_W / TM_W)
        by2 = b4[0] * (FM_H / TM_H)
        bcx = (bx1 + bx2) / 2.0
        bcy = (by1 + by2) / 2.0
        bw = bx2 - bx1
        bh = by2 - by1
        tx_ref[0] = (bcx - acx) / aw
        ty_ref[0] = (bcy - acy) / ah
        tw_ref[0] = jnp.log(bw / aw)
        th_ref[0] = jnp.log(bh / ah)
        S_p = jnp.minimum(cnt[0], num_pos)
        pos_iota = lax.broadcasted_iota(jnp.int32, (1, A), 1)
        obj_ref[0] = (pos_iota < S_p).astype(jnp.float32)

    return body


def _run_offsets(B, A, num_pos, gcomps, b0cols, cnt_col):
    return pl.pallas_call(
        _make_offsets_body(num_pos, A),
        grid=(B,),
        in_specs=[pl.BlockSpec((1, 1, A), lambda b: (b, 0, 0))] * 4
        + [pl.BlockSpec((1, 1, 1), lambda b: (b, 0, 0))] * 5,
        out_specs=[pl.BlockSpec((1, 1, A), lambda b: (b, 0, 0))] * 5,
        out_shape=[jax.ShapeDtypeStruct((B, 1, A), jnp.float32)] * 5,
    )(*[g[:, None, :] for g in gcomps], *[c[:, :, None] for c in b0cols],
      cnt_col[:, :, None])


def kernel(anchors, gt_bboxes, gt_orig_classes):
    B, A, _ = anchors.shape
    G = gt_bboxes.shape[1]
    N = A * G
    num_neg = int(RATIO * A)
    num_pos = A - num_neg
    negcap = A + LN

    comps = [anchors[..., i] for i in range(4)]
    gcols = [gt_bboxes[..., i][..., None] for i in range(4)]

    comps3 = [c[:, None, :] for c in comps]
    iou_t, code_t = _run_iou_code(B, A, G, comps3, gcols)
    iou_flat = iou_t.transpose(0, 2, 1).reshape(B, N)
    code_flat = code_t.transpose(0, 2, 1).reshape(B, N)

    k2 = _make_compact(B, N, negcap)
    pos_iou, pos_idx, neg_idx, counts = k2(
        iou_flat.reshape(-1), code_flat.reshape(-1))
    pos_iou = pos_iou.reshape(B, PCAP)
    pos_idx = pos_idx.reshape(B, PCAP)
    neg_idx = neg_idx.reshape(B, negcap)
    counts = counts.reshape(B, LN)

    ranks = _run_rank(B, pos_iou, counts)

    k4 = _make_scatter_gather(B, A, G, negcap, num_pos)
    g1, g2, g3, g4 = (g.reshape(B, A) for g in k4(
        pos_idx.reshape(-1), ranks.reshape(-1), counts.reshape(-1),
        neg_idx.reshape(-1), *[c.reshape(-1) for c in comps]))

    b0cols = [gt_bboxes[:, 0, i][:, None] for i in range(4)]
    cnt_col = counts[:, 0:1]
    tx, ty, tw, th, obj = (o[:, 0, :] for o in _run_offsets(
        B, A, num_pos, (g1, g2, g3, g4), b0cols, cnt_col))

    all_anchors = jnp.stack([g1, g2, g3, g4], axis=-1)
    gt_conf = obj[..., None]
    gt_cls = jnp.broadcast_to(
        gt_orig_classes[:, 0][:, None, None], (B, A, 1)
    ).astype(gt_orig_classes.dtype)
    gt_off = jnp.stack([tx, ty, tw, th], axis=-1)
    return all_anchors, gt_conf, gt_cls, gt_off
